# speedup vs baseline: 3.3140x; 3.3140x over previous
import functools

import jax
import jax.numpy as jnp
from jax import lax
from jax.experimental import pallas as pl
from jax.experimental.pallas import tpu as pltpu
from jax.experimental.pallas import tpu_sc as plsc

_N = 8192
_B = 16384


_R = 512


def _lse_block(w_ref, out_ref):
    x = w_ref[...]
    m = jnp.max(x, axis=1, keepdims=True)
    s = jnp.sum(jnp.exp(x - m), axis=1)
    out_ref[...] = jnp.log(s) + m[:, 0]


def _row_lse(w):
    return pl.pallas_call(
        _lse_block,
        grid=(_N // _R,),
        in_specs=[pl.BlockSpec((_R, _N), lambda i: (i, 0))],
        out_specs=pl.BlockSpec((_R,), lambda i: (i,)),
        out_shape=jax.ShapeDtypeStruct((_N,), jnp.float32),
    )(w)


_NC, _NS, _L = 2, 16, 16
_NW = _NC * _NS
_BPW = _B // _NW
_CH = 128
_NCH = _BPW // _CH


def _sc_body(w_flat, lse, conds3, inp3, out3, c_v, i_v, idx_v, val_v, lseg_v, o_v, sem):
    wid = lax.axis_index("s") * _NC + lax.axis_index("c")
    pltpu.sync_copy(conds3.at[wid], c_v)
    pltpu.sync_copy(inp3.at[wid], i_v)
    for j in range(_NCH):
        for t in range(_CH // _L):
            sl = pl.ds(t * _L, _L)
            idx_v[j, sl] = c_v[j, sl] * _N + i_v[j, sl]
    for j in range(_NCH):
        pltpu.async_copy(w_flat.at[idx_v.at[j]], val_v.at[j], sem).wait()
        pltpu.async_copy(lse.at[c_v.at[j]], lseg_v.at[j], sem).wait()
    for j in range(_NCH):
        for t in range(_CH // _L):
            sl = pl.ds(t * _L, _L)
            o_v[j, sl] = val_v[j, sl] - lseg_v[j, sl]
    pltpu.sync_copy(o_v, out3.at[wid])


def _sc_gather(w_flat, lse, conds3, inp3):
    mesh = plsc.VectorSubcoreMesh(core_axis_name="c", subcore_axis_name="s")
    return pl.kernel(
        _sc_body,
        out_type=jax.ShapeDtypeStruct((_NW, _NCH, _CH), jnp.float32),
        mesh=mesh,
        scratch_types=[
            pltpu.VMEM((_NCH, _CH), jnp.int32),
            pltpu.VMEM((_NCH, _CH), jnp.int32),
            pltpu.VMEM((_NCH, _CH), jnp.int32),
            pltpu.VMEM((_NCH, _CH), jnp.float32),
            pltpu.VMEM((_NCH, _CH), jnp.float32),
            pltpu.VMEM((_NCH, _CH), jnp.float32),
            pltpu.SemaphoreType.DMA,
        ],
    )(w_flat, lse, conds3, inp3)


def kernel(inputs, conds, w):
    conds_ = conds.reshape(_NW, _NCH, _CH).astype(jnp.int32)
    inp_ = inputs.reshape(_NW, _NCH, _CH).astype(jnp.int32)
    lse = _row_lse(w)
    out3 = _sc_gather(w.reshape(-1), lse, conds_, inp_)
    return out3.reshape(_B)
